# baseline (speedup 1.0000x reference)
import functools

import jax
import jax.numpy as jnp
from jax import lax
from jax.experimental import pallas as pl
from jax.experimental.pallas import tpu as pltpu
from jax.experimental.pallas import tpu_sc as plsc

_NUM_CORES = 2
_NUM_SUBCORES = 16
_NUM_WORKERS = _NUM_CORES * _NUM_SUBCORES
_LANES = 16
_PACK = 4


def _sc_gather_wide(table_wide, idx):
    batch = idx.shape[0]
    wide_dim = table_wide.shape[1]
    b_per_w = batch // _NUM_WORKERS
    mesh = plsc.VectorSubcoreMesh(core_axis_name="c", subcore_axis_name="s")

    @functools.partial(
        pl.kernel, mesh=mesh,
        out_type=jax.ShapeDtypeStruct((batch, wide_dim), table_wide.dtype),
        scratch_types=[
            pltpu.VMEM((b_per_w,), jnp.int32),
            pltpu.VMEM((b_per_w,), jnp.int32),
            pltpu.VMEM((b_per_w, wide_dim), table_wide.dtype),
            pltpu.SemaphoreType.DMA,
        ],
    )
    def gather_kernel(table_hbm, idx_hbm, wide_hbm, idx_v, row_v, rows_v, sem):
        wid = lax.axis_index("s") * _NUM_CORES + lax.axis_index("c")
        base = wid * b_per_w
        pltpu.sync_copy(idx_hbm.at[pl.ds(base, b_per_w)], idx_v)

        @pl.loop(0, b_per_w, step=_LANES)
        def _(i):
            row_v[pl.ds(i, _LANES)] = jax.lax.shift_right_logical(
                idx_v[pl.ds(i, _LANES)], 2)

        pltpu.async_copy(table_hbm.at[row_v], rows_v, sem).wait()
        pltpu.sync_copy(rows_v, wide_hbm.at[pl.ds(base, b_per_w)])

    return gather_kernel(table_wide, idx)


def _tc_extract(wide, idx2d, dim):
    batch, wide_dim = wide.shape
    block = 2048
    grid = batch // block

    def extract_kernel(wide_ref, idx_ref, out_ref):
        sel = idx_ref[...] & (_PACK - 1)
        w = wide_ref[...]
        acc = jnp.zeros((block, dim), wide.dtype)
        for g in range(_PACK):
            acc = jnp.where(sel == g, w[:, g * dim:(g + 1) * dim], acc)
        out_ref[...] = acc

    return pl.pallas_call(
        extract_kernel,
        grid=(grid,),
        in_specs=[
            pl.BlockSpec((block, wide_dim), lambda i: (i, 0)),
            pl.BlockSpec((block, 1), lambda i: (i, 0)),
        ],
        out_specs=pl.BlockSpec((block, dim), lambda i: (i, 0)),
        out_shape=jax.ShapeDtypeStruct((batch, dim), wide.dtype),
        compiler_params=pltpu.CompilerParams(
            dimension_semantics=("parallel",)),
    )(wide, idx2d)


def kernel(user_indices, table):
    num_rows, dim = table.shape
    idx = user_indices.astype(jnp.int32)
    table_wide = table.reshape(num_rows // _PACK, dim * _PACK)
    wide = _sc_gather_wide(table_wide, idx)
    return _tc_extract(wide, idx.reshape(-1, 1), dim)


# speedup vs baseline: 3.7031x; 3.7031x over previous
import functools

import jax
import jax.numpy as jnp
from jax import lax
from jax.experimental import pallas as pl
from jax.experimental.pallas import tpu as pltpu
from jax.experimental.pallas import tpu_sc as plsc

_NUM_CORES = 2
_NUM_SUBCORES = 16
_NUM_WORKERS = _NUM_CORES * _NUM_SUBCORES
_LANES = 16
_WIN = 128
_CHUNK = 16


def kernel(user_indices, table):
    batch = user_indices.shape[0]
    dim = table.shape[1]
    b_per_w = batch // _NUM_WORKERS
    idx = user_indices.astype(jnp.int32)
    tT = table.T

    mesh = plsc.VectorSubcoreMesh(core_axis_name="c", subcore_axis_name="s")

    @functools.partial(
        pl.kernel, mesh=mesh,
        out_type=jax.ShapeDtypeStruct((dim, batch), table.dtype),
        scratch_types=[
            pltpu.VMEM((b_per_w,), jnp.int32),
            pltpu.VMEM((_CHUNK, dim, _WIN), table.dtype),
            pltpu.VMEM((dim, b_per_w), table.dtype),
            pltpu.SemaphoreType.DMA,
        ],
        compiler_params=pltpu.CompilerParams(disable_bounds_checks=True,
                                             needs_layout_passes=False),
    )
    def gather_kernel(tT_hbm, idx_hbm, outT_hbm, idx_v, win_v, acc_v, sem):
        wid = lax.axis_index("s") * _NUM_CORES + lax.axis_index("c")
        base = wid * b_per_w
        pltpu.sync_copy(idx_hbm.at[pl.ds(base, b_per_w)], idx_v)

        row_lo = lax.iota(jnp.int32, _LANES)
        row_hi = row_lo + _LANES

        @pl.loop(0, b_per_w, step=_CHUNK)
        def _(i):
            v = idx_v[pl.ds(i, _CHUNK)]
            handles = []
            for m in range(_CHUNK):
                s = v[m]
                soff = pl.multiple_of((s >> 7) << 7, _WIN)
                handles.append(pltpu.async_copy(
                    tT_hbm.at[:, pl.ds(soff, _WIN)], win_v.at[m], sem))
            for h in handles:
                h.wait()
            for m in range(_CHUNK):
                c = v[m] & (_WIN - 1)
                cvec = jnp.zeros((_LANES,), jnp.int32) + c
                kvec = jnp.zeros((_LANES,), jnp.int32) + (i + m)
                g0 = plsc.load_gather(win_v.at[m], [row_lo, cvec])
                g1 = plsc.load_gather(win_v.at[m], [row_hi, cvec])
                plsc.store_scatter(acc_v, [row_lo, kvec], g0)
                plsc.store_scatter(acc_v, [row_hi, kvec], g1)

        pltpu.sync_copy(acc_v, outT_hbm.at[:, pl.ds(base, b_per_w)])

    outT = gather_kernel(tT, idx)
    return outT.T


# speedup vs baseline: 3.9525x; 1.0673x over previous
import functools

import jax
import jax.numpy as jnp
from jax import lax
from jax.experimental import pallas as pl
from jax.experimental.pallas import tpu as pltpu
from jax.experimental.pallas import tpu_sc as plsc

_NUM_CORES = 2
_NUM_SUBCORES = 16
_NUM_WORKERS = _NUM_CORES * _NUM_SUBCORES
_LANES = 16
_WIN = 128
_CHUNK = 8


def kernel(user_indices, table):
    batch = user_indices.shape[0]
    dim = table.shape[1]
    b_per_w = batch // _NUM_WORKERS
    idx = user_indices.astype(jnp.int32)
    tT = table.T

    mesh = plsc.VectorSubcoreMesh(core_axis_name="c", subcore_axis_name="s")

    @functools.partial(
        pl.kernel, mesh=mesh,
        out_type=jax.ShapeDtypeStruct((dim, batch), table.dtype),
        scratch_types=[
            pltpu.VMEM((b_per_w,), jnp.int32),
            pltpu.VMEM((2 * _CHUNK, dim, _WIN), table.dtype),
            pltpu.VMEM((dim, b_per_w), table.dtype),
            pltpu.SemaphoreType.DMA,
            pltpu.SemaphoreType.DMA,
        ],
        compiler_params=pltpu.CompilerParams(disable_bounds_checks=True,
                                             needs_layout_passes=False),
    )
    def gather_kernel(tT_hbm, idx_hbm, outT_hbm, idx_v, win_v, acc_v,
                      sem0, sem1):
        wid = lax.axis_index("s") * _NUM_CORES + lax.axis_index("c")
        base = wid * b_per_w
        pltpu.sync_copy(idx_hbm.at[pl.ds(base, b_per_w)], idx_v)

        row_lo = lax.iota(jnp.int32, _LANES)
        row_hi = row_lo + _LANES
        sems = (sem0, sem1)

        def fire(v, half):
            for m in range(_CHUNK):
                s = v[half * _CHUNK + m]
                soff = pl.multiple_of((s >> 7) << 7, _WIN)
                pltpu.async_copy(
                    tT_hbm.at[:, pl.ds(soff, _WIN)],
                    win_v.at[half * _CHUNK + m], sems[half])

        def drain_and_extract(v, i, half):
            for m in range(_CHUNK):
                pltpu.make_async_copy(
                    tT_hbm.at[:, pl.ds(0, _WIN)],
                    win_v.at[half * _CHUNK + m], sems[half]).wait()
            for m in range(_CHUNK):
                c = v[half * _CHUNK + m] & (_WIN - 1)
                cvec = jnp.zeros((_LANES,), jnp.int32) + c
                kvec = jnp.zeros((_LANES,), jnp.int32) + (i + half * _CHUNK + m)
                g0 = plsc.load_gather(win_v.at[half * _CHUNK + m],
                                      [row_lo, cvec])
                g1 = plsc.load_gather(win_v.at[half * _CHUNK + m],
                                      [row_hi, cvec])
                plsc.store_scatter(acc_v, [row_lo, kvec], g0)
                plsc.store_scatter(acc_v, [row_hi, kvec], g1)

        v0 = idx_v[pl.ds(0, 2 * _CHUNK)]
        fire(v0, 0)

        @pl.loop(0, b_per_w, step=2 * _CHUNK)
        def _(i):
            v = idx_v[pl.ds(i, 2 * _CHUNK)]
            fire(v, 1)
            drain_and_extract(v, i, 0)

            @pl.when(i + 2 * _CHUNK < b_per_w)
            def _():
                vn = idx_v[pl.ds(i + 2 * _CHUNK, 2 * _CHUNK)]
                fire(vn, 0)

            drain_and_extract(v, i, 1)

        pltpu.sync_copy(acc_v, outT_hbm.at[:, pl.ds(base, b_per_w)])

    outT = gather_kernel(tT, idx)
    return outT.T
